# baseline (speedup 1.0000x reference)
import functools

import jax
import jax.numpy as jnp
from jax import lax
from jax.experimental import pallas as pl
from jax.experimental.pallas import tpu as pltpu
from jax.experimental.pallas import tpu_sc as plsc

N, E, D = 10000, 160000, 256
HALF = D // 2
NTILES = 16
NP = 10112
RPT = NP // NTILES
EPT = E // NTILES
B = 128
NB = 80
NCHUNK = 2
CB = NB // NCHUNK

_mesh = plsc.VectorSubcoreMesh(core_axis_name="c", subcore_axis_name="s")


@functools.partial(
    pl.kernel,
    out_type=jax.ShapeDtypeStruct((2 * NP, HALF), jnp.float32),
    mesh=_mesh,
    scratch_types=[
        pltpu.VMEM((CB, B), jnp.int32),
        pltpu.VMEM((CB, B), jnp.int32),
        pltpu.VMEM((B, HALF), jnp.float32),
        pltpu.VMEM((B, HALF), jnp.float32),
        pltpu.VMEM_SHARED((NP, HALF), jnp.float32),
        pltpu.SemaphoreType.DMA,
        pltpu.SemaphoreType.DMA,
        pltpu.SemaphoreType.DMA,
        pltpu.SemaphoreType.DMA,
    ],
)
def _aggregate(xcat, srcg, dstg, out, src_v, dst_v, buf0, buf1, acc,
               gsem0, gsem1, ssem0, ssem1):
    c = lax.axis_index("c")
    s = lax.axis_index("s")
    r0 = pl.multiple_of(s * RPT, 8)
    base = pl.multiple_of(c * NP, 8)

    off = 0
    for sz in (128, 128, 128, 128, RPT - 512):
        pltpu.sync_copy(xcat.at[pl.ds(base + r0 + off, sz)],
                        acc.at[pl.ds(r0 + off, sz)])
        off += sz

    plsc.subcore_barrier()

    ring = ((buf0, gsem0, ssem0), (buf1, gsem1, ssem1))

    def g_start(k, buf, sem):
        pltpu.async_copy(xcat.at[src_v.at[k]], buf, sem)

    def g_wait(k, buf, sem):
        pltpu.make_async_copy(xcat.at[src_v.at[k]], buf, sem).wait()

    def s_start(k, buf, sem):
        pltpu.async_copy(buf, acc.at[dst_v.at[k]], sem, add=True)

    def s_wait(k, buf, sem):
        pltpu.make_async_copy(buf, acc.at[dst_v.at[k]], sem).wait()

    for cc in range(NCHUNK):
        pltpu.sync_copy(srcg.at[c * NTILES + s].at[pl.ds(cc * CB, CB)], src_v)
        pltpu.sync_copy(dstg.at[s].at[pl.ds(cc * CB, CB)], dst_v)

        g_start(0, buf0, gsem0)

        @pl.loop(0, CB // 2)
        def _pair(gp):
            for b in range(2):
                buf, gsem, ssem = ring[b]
                nbuf, ngsem, nssem = ring[1 - b]
                k = gp * 2 + b
                g_wait(k, buf, gsem)
                s_start(k, buf, ssem)
                nk = k + 1

                @pl.when(jnp.logical_and(nk < CB, nk >= 2))
                def _():
                    s_wait(nk - 2, nbuf, nssem)

                @pl.when(nk < CB)
                def _():
                    g_start(nk, nbuf, ngsem)

        s_wait(CB - 2, ring[(CB - 2) % 2][0], ring[(CB - 2) % 2][2])
        s_wait(CB - 1, ring[(CB - 1) % 2][0], ring[(CB - 1) % 2][2])

    plsc.subcore_barrier()
    pltpu.sync_copy(acc.at[pl.ds(r0, RPT)], out.at[pl.ds(base + r0, RPT)])


def _mlp_body(s_ref, x_ref, w1_ref, b1_ref, w2_ref, b2_ref, o_ref):
    dn = (((1,), (0,)), ((), ()))
    h = jnp.concatenate([s_ref[0], s_ref[1]], axis=1)
    z = lax.dot_general(h, w1_ref[...], dn,
                        precision=lax.Precision.HIGHEST,
                        preferred_element_type=jnp.float32) + b1_ref[...]
    sp = jnp.maximum(z, 0.0) + jnp.log1p(jnp.exp(-jnp.abs(z)))
    h1 = z * jnp.tanh(sp)
    z2 = lax.dot_general(h1, w2_ref[...], dn,
                         precision=lax.Precision.HIGHEST,
                         preferred_element_type=jnp.float32) + b2_ref[...]
    o_ref[...] = x_ref[...] + z2


_BM = 1000


def kernel(x, edge_index, W1, b1, W2, b2):
    src = edge_index[0]
    dst = edge_index[1]

    xp = jnp.pad(x, ((0, NP - N), (0, 0)))
    xcat = xp.reshape(NP, 2, HALF).transpose(1, 0, 2).reshape(2 * NP, HALF)

    srcp = jnp.pad(src.reshape(NTILES, EPT), ((0, 0), (0, NB * B - EPT)))
    srcg = (srcp[None] + (jnp.arange(2, dtype=jnp.int32) * NP)[:, None, None])
    srcg = srcg.reshape(2 * NTILES, NB, B)
    dstg = jnp.pad(dst.reshape(NTILES, EPT), ((0, 0), (0, NB * B - EPT)),
                   constant_values=N).reshape(NTILES, NB, B)

    s3 = _aggregate(xcat, srcg, dstg).reshape(2, NP, HALF)

    return pl.pallas_call(
        _mlp_body,
        grid=(N // _BM,),
        in_specs=[
            pl.BlockSpec((2, _BM, HALF), lambda i: (0, i, 0)),
            pl.BlockSpec((_BM, D), lambda i: (i, 0)),
            pl.BlockSpec((D, D), lambda i: (0, 0)),
            pl.BlockSpec((1, D), lambda i: (0, 0)),
            pl.BlockSpec((D, D), lambda i: (0, 0)),
            pl.BlockSpec((1, D), lambda i: (0, 0)),
        ],
        out_specs=pl.BlockSpec((_BM, D), lambda i: (i, 0)),
        out_shape=jax.ShapeDtypeStruct((N, D), jnp.float32),
    )(s3, x, W1, b1.reshape(1, D), W2, b2.reshape(1, D))


# speedup vs baseline: 1.0681x; 1.0681x over previous
import functools

import jax
import jax.numpy as jnp
from jax import lax
from jax.experimental import pallas as pl
from jax.experimental.pallas import tpu as pltpu
from jax.experimental.pallas import tpu_sc as plsc

N, E, D = 10000, 160000, 256
HALF = D // 2
NTILES = 16
NP = 10112
RPT = NP // NTILES
EPT = E // NTILES
B = 128
NB = 80
NCHUNK = 2
CB = NB // NCHUNK

_mesh = plsc.VectorSubcoreMesh(core_axis_name="c", subcore_axis_name="s")


@functools.partial(
    pl.kernel,
    out_type=jax.ShapeDtypeStruct((2 * NP, HALF), jnp.float32),
    mesh=_mesh,
    scratch_types=[
        pltpu.VMEM((CB, B), jnp.int32),
        pltpu.VMEM((CB, B), jnp.int32),
        pltpu.VMEM((B, HALF), jnp.float32),
        pltpu.VMEM((B, HALF), jnp.float32),
        pltpu.VMEM_SHARED((NP, HALF), jnp.float32),
        pltpu.SemaphoreType.DMA,
        pltpu.SemaphoreType.DMA,
        pltpu.SemaphoreType.DMA,
        pltpu.SemaphoreType.DMA,
    ],
)
def _aggregate(xcat, srcg, dstg, out, src_v, dst_v, buf0, buf1, acc,
               gsem0, gsem1, ssem0, ssem1):
    c = lax.axis_index("c")
    s = lax.axis_index("s")
    r0 = pl.multiple_of(s * RPT, 8)
    base = pl.multiple_of(c * NP, 8)

    off = 0
    for sz in (128, 128, 128, 128, RPT - 512):
        pltpu.sync_copy(xcat.at[pl.ds(base + r0 + off, sz)],
                        acc.at[pl.ds(r0 + off, sz)])
        off += sz

    plsc.subcore_barrier()

    ring = ((buf0, gsem0, ssem0), (buf1, gsem1, ssem1))

    GSPLIT = 4

    def g_start(k, buf, sem):
        idx = src_v.at[k]
        w = B // GSPLIT
        for j in range(GSPLIT):
            pltpu.async_copy(xcat.at[idx.at[pl.ds(j * w, w)]],
                             buf.at[pl.ds(j * w, w)], sem)

    def g_wait(k, buf, sem):
        pltpu.make_async_copy(xcat.at[src_v.at[k]], buf, sem).wait()

    def s_start(k, buf, sem):
        pltpu.async_copy(buf, acc.at[dst_v.at[k]], sem, add=True)

    def s_wait(k, buf, sem):
        pltpu.make_async_copy(buf, acc.at[dst_v.at[k]], sem).wait()

    for cc in range(NCHUNK):
        pltpu.sync_copy(srcg.at[c * NTILES + s].at[pl.ds(cc * CB, CB)], src_v)
        pltpu.sync_copy(dstg.at[s].at[pl.ds(cc * CB, CB)], dst_v)

        g_start(0, buf0, gsem0)
        g_start(1, buf1, gsem1)

        @pl.loop(0, CB // 2)
        def _pair(gp):
            for b in range(2):
                buf, gsem, ssem = ring[b]
                k = gp * 2 + b
                g_wait(k, buf, gsem)
                pltpu.sync_copy(buf, acc.at[dst_v.at[k]], add=True)
                nk = k + 2

                @pl.when(nk < CB)
                def _():
                    g_start(nk, buf, gsem)

    plsc.subcore_barrier()
    pltpu.sync_copy(acc.at[pl.ds(r0, RPT)], out.at[pl.ds(base + r0, RPT)])


def _mlp_body(s_ref, x_ref, w1_ref, b1_ref, w2_ref, b2_ref, o_ref):
    dn = (((1,), (0,)), ((), ()))
    h = jnp.concatenate([s_ref[0], s_ref[1]], axis=1)
    z = lax.dot_general(h, w1_ref[...], dn,
                        precision=lax.Precision.HIGHEST,
                        preferred_element_type=jnp.float32) + b1_ref[...]
    sp = jnp.maximum(z, 0.0) + jnp.log1p(jnp.exp(-jnp.abs(z)))
    h1 = z * jnp.tanh(sp)
    z2 = lax.dot_general(h1, w2_ref[...], dn,
                         precision=lax.Precision.HIGHEST,
                         preferred_element_type=jnp.float32) + b2_ref[...]
    o_ref[...] = x_ref[...] + z2


_BM = 1000


def kernel(x, edge_index, W1, b1, W2, b2):
    src = edge_index[0]
    dst = edge_index[1]

    xp = jnp.pad(x, ((0, NP - N), (0, 0)))
    xcat = xp.reshape(NP, 2, HALF).transpose(1, 0, 2).reshape(2 * NP, HALF)

    srcp = jnp.pad(src.reshape(NTILES, EPT), ((0, 0), (0, NB * B - EPT)))
    srcg = (srcp[None] + (jnp.arange(2, dtype=jnp.int32) * NP)[:, None, None])
    srcg = srcg.reshape(2 * NTILES, NB, B)
    dstg = jnp.pad(dst.reshape(NTILES, EPT), ((0, 0), (0, NB * B - EPT)),
                   constant_values=N).reshape(NTILES, NB, B)

    s3 = _aggregate(xcat, srcg, dstg).reshape(2, NP, HALF)

    return pl.pallas_call(
        _mlp_body,
        grid=(N // _BM,),
        in_specs=[
            pl.BlockSpec((2, _BM, HALF), lambda i: (0, i, 0)),
            pl.BlockSpec((_BM, D), lambda i: (i, 0)),
            pl.BlockSpec((D, D), lambda i: (0, 0)),
            pl.BlockSpec((1, D), lambda i: (0, 0)),
            pl.BlockSpec((D, D), lambda i: (0, 0)),
            pl.BlockSpec((1, D), lambda i: (0, 0)),
        ],
        out_specs=pl.BlockSpec((_BM, D), lambda i: (i, 0)),
        out_shape=jax.ShapeDtypeStruct((N, D), jnp.float32),
    )(s3, x, W1, b1.reshape(1, D), W2, b2.reshape(1, D))


# speedup vs baseline: 1.1583x; 1.0845x over previous
import functools

import jax
import jax.numpy as jnp
from jax import lax
from jax.experimental import pallas as pl
from jax.experimental.pallas import tpu as pltpu
from jax.experimental.pallas import tpu_sc as plsc

N, E, D = 10000, 160000, 256
HALF = D // 2
NTILES = 16
NP = 10112
RPT = NP // NTILES
EPT = E // NTILES
B = 128
NB = 80
NCHUNK = 2
CB = NB // NCHUNK

_mesh = plsc.VectorSubcoreMesh(core_axis_name="c", subcore_axis_name="s")


@functools.partial(
    pl.kernel,
    out_type=jax.ShapeDtypeStruct((2 * NP, HALF), jnp.float32),
    mesh=_mesh,
    scratch_types=[
        pltpu.VMEM((CB, B), jnp.int32),
        pltpu.VMEM((CB, B), jnp.int32),
        pltpu.VMEM((B, HALF), jnp.float32),
        pltpu.VMEM((B, HALF), jnp.float32),
        pltpu.VMEM_SHARED((NP, HALF), jnp.float32),
        pltpu.SemaphoreType.DMA,
        pltpu.SemaphoreType.DMA,
    ],
)
def _aggregate(xview, srcg, dstg, out, src_v, dst_v, buf0, buf1, acc,
               gsem0, gsem1):
    c = lax.axis_index("c")
    s = lax.axis_index("s")
    r0 = pl.multiple_of(s * RPT, 8)
    base = pl.multiple_of(c * NP, 8)

    zeros = jnp.zeros((16,), jnp.float32)

    @pl.loop(0, B)
    def _zrow(r):
        for j in range(HALF // 16):
            buf0[r, pl.ds(j * 16, 16)] = zeros

    for kk in range(RPT // B + 1):
        sz = B if kk < RPT // B else RPT - (RPT // B) * B
        pltpu.sync_copy(buf0.at[pl.ds(0, sz)], acc.at[pl.ds(r0 + kk * B, sz)])

    plsc.subcore_barrier()

    ring = ((buf0, gsem0), (buf1, gsem1))
    GSPLIT = 4

    def g_start(k, buf, sem):
        idx = src_v.at[k]
        w = B // GSPLIT
        for j in range(GSPLIT):
            pltpu.async_copy(xview.at[idx.at[pl.ds(j * w, w)]],
                             buf.at[pl.ds(j * w, w)], sem)

    def g_wait(k, buf, sem):
        pltpu.make_async_copy(xview.at[src_v.at[k]], buf, sem).wait()

    for cc in range(NCHUNK):
        pltpu.sync_copy(srcg.at[c * NTILES + s].at[pl.ds(cc * CB, CB)], src_v)
        pltpu.sync_copy(dstg.at[s].at[pl.ds(cc * CB, CB)], dst_v)

        g_start(0, buf0, gsem0)
        g_start(1, buf1, gsem1)

        @pl.loop(0, CB // 2)
        def _pair(gp):
            for b in range(2):
                buf, gsem = ring[b]
                k = gp * 2 + b
                g_wait(k, buf, gsem)
                pltpu.sync_copy(buf, acc.at[dst_v.at[k]], add=True)
                nk = k + 2

                @pl.when(nk < CB)
                def _():
                    g_start(nk, buf, gsem)

    plsc.subcore_barrier()
    pltpu.sync_copy(acc.at[pl.ds(r0, RPT)], out.at[pl.ds(base + r0, RPT)])


def _mlp_body(s_ref, x_ref, w1_ref, b1_ref, w2_ref, b2_ref, o_ref):
    dn = (((1,), (0,)), ((), ()))
    xb = x_ref[...]
    h = xb + jnp.concatenate([s_ref[0], s_ref[1]], axis=1)
    z = lax.dot_general(h.astype(jnp.bfloat16), w1_ref[...], dn,
                        preferred_element_type=jnp.float32) + b1_ref[...]
    u = 1.0 + jnp.exp(jnp.minimum(z, 40.0))
    h1 = z * (1.0 - 2.0 / (u * u + 1.0))
    z2 = lax.dot_general(h1.astype(jnp.bfloat16), w2_ref[...], dn,
                         preferred_element_type=jnp.float32) + b2_ref[...]
    o_ref[...] = xb + z2


_BM = 1000


def kernel(x, edge_index, W1, b1, W2, b2):
    src = edge_index[0]
    dst = edge_index[1]

    xview = x.reshape(2 * N, HALF)

    srcp = jnp.pad(src.reshape(NTILES, EPT), ((0, 0), (0, NB * B - EPT)))
    srcg = (2 * srcp[None] + jnp.arange(2, dtype=jnp.int32)[:, None, None])
    srcg = srcg.reshape(2 * NTILES, NB, B)
    dstg = jnp.pad(dst.reshape(NTILES, EPT), ((0, 0), (0, NB * B - EPT)),
                   constant_values=N).reshape(NTILES, NB, B)

    s3 = _aggregate(xview, srcg, dstg).reshape(2, NP, HALF)

    return pl.pallas_call(
        _mlp_body,
        grid=(N // _BM,),
        in_specs=[
            pl.BlockSpec((2, _BM, HALF), lambda i: (0, i, 0)),
            pl.BlockSpec((_BM, D), lambda i: (i, 0)),
            pl.BlockSpec((D, D), lambda i: (0, 0)),
            pl.BlockSpec((1, D), lambda i: (0, 0)),
            pl.BlockSpec((D, D), lambda i: (0, 0)),
            pl.BlockSpec((1, D), lambda i: (0, 0)),
        ],
        out_specs=pl.BlockSpec((_BM, D), lambda i: (i, 0)),
        out_shape=jax.ShapeDtypeStruct((N, D), jnp.float32),
    )(s3, x, W1.astype(jnp.bfloat16), b1.reshape(1, D),
      W2.astype(jnp.bfloat16), b2.reshape(1, D))
